# baseline (speedup 1.0000x reference)
import functools

import jax
import jax.numpy as jnp
from jax import lax
from jax.experimental import pallas as pl
from jax.experimental.pallas import tpu as pltpu
from jax.experimental.pallas import tpu_sc as plsc

N = 10000
D = 512
B = 256
NS = 256
ES = 2048
THRESHOLD = 0.1
DW = D // 2
NPAD = 10240
KBLK = 2048
RB = 512
NC = 2
NSUB = 16
NW = NC * NSUB
NPW = B // NW
GCH = 64
IOFF = 16
NCHUNK = 2


def _norm_body(f_ref, t_ref, v_ref, local_ref, dtab_ref):
    i = pl.program_id(0)
    f = f_ref[...]
    t = t_ref[...]
    v = v_ref[...]
    eps = 1e-12
    fn = f / jnp.maximum(jnp.sqrt(jnp.sum(f * f, axis=1, keepdims=True)), eps)
    en = t / jnp.maximum(jnp.sqrt(jnp.sum(t * t, axis=1, keepdims=True)), eps)
    lc = en - v
    lcn = lc / jnp.maximum(jnp.sqrt(jnp.sum(lc * lc, axis=1, keepdims=True)), eps)
    lcb = lcn.astype(jnp.bfloat16)
    lo = lax.bitcast_convert_type(lcb[:, :DW], jnp.uint16).astype(jnp.uint32)
    hi = lax.bitcast_convert_type(lcb[:, DW:], jnp.uint16).astype(jnp.uint32)
    local_ref[...] = (lo | (hi << 16)).astype(jnp.int32)
    rid = i * RB + lax.broadcasted_iota(jnp.int32, (RB, 1), 0)
    dtab_ref[...] = jnp.where(rid < N, fn - en, 0.0)


def _normalize(feature, text_embedding, virtual):
    grid = NPAD // RB
    return pl.pallas_call(
        _norm_body,
        grid=(grid,),
        in_specs=[
            pl.BlockSpec((RB, D), lambda i: (i, 0)),
            pl.BlockSpec((RB, D), lambda i: (i, 0)),
            pl.BlockSpec((1, D), lambda i: (0, 0)),
        ],
        out_specs=[
            pl.BlockSpec((RB, DW), lambda i: (i, 0)),
            pl.BlockSpec((RB, D), lambda i: (i, 0)),
        ],
        out_shape=[
            jax.ShapeDtypeStruct((N, DW), jnp.int32),
            jax.ShapeDtypeStruct((NPAD, D), jnp.float32),
        ],
    )(feature, text_embedding, virtual)


def _sc_body(b0, npw, local_hbm, nidx_hbm, ee_hbm,
             teb_hbm, adj_hbm, cnt_hbm,
             idx_v, rows_v, er_v, ec_v, adjbuf, cb1, cb2, sem):
    wid = lax.axis_index("s") * NC + lax.axis_index("c")

    z16f = jnp.zeros((16,), jnp.float32)
    one16f = jnp.ones((16,), jnp.float32)
    io16f = lax.iota(jnp.int32, 16).astype(jnp.float32)

    def _z_adj(r, _):
        for cc in range(NS // 16):
            adjbuf[r, pl.ds(cc * 16, 16)] = z16f
        return _
    lax.fori_loop(0, NS, _z_adj, None)

    def _z_cb(i, _):
        cb1[pl.ds(i * 16, 16)] = z16f
        cb2[pl.ds(i * 16, 16)] = z16f
        return _
    lax.fori_loop(0, NPAD // 16, _z_cb, None)

    def _node(j, _):
        bl = wid * npw + j
        b = b0 + bl

        idx_v[pl.ds(0, 16)] = jnp.full((16,), -1, jnp.int32)
        idx_v[pl.ds(IOFF + NS, 16)] = jnp.full((16,), 0x40000000, jnp.int32)
        pltpu.sync_copy(nidx_hbm.at[pl.ds(b * NS, NS)],
                        idx_v.at[pl.ds(IOFF, NS)])
        pltpu.sync_copy(ee_hbm.at[pl.ds((2 * b) * ES, ES)], er_v)
        pltpu.sync_copy(ee_hbm.at[pl.ds((2 * b + 1) * ES, ES)], ec_v)

        for kk in range(NS // GCH):
            pltpu.async_copy(
                local_hbm.at[idx_v.at[pl.ds(IOFF + kk * GCH, GCH)]],
                rows_v, sem).wait()
            pltpu.sync_copy(rows_v, teb_hbm.at[pl.ds(bl * NS + kk * GCH, GCH)])

        def _runs(k, _):
            v = idx_v[pl.ds(IOFF + k * 16, 16)]
            vp = idx_v[pl.ds(IOFF - 1 + k * 16, 16)]
            vn = idx_v[pl.ds(IOFF + 1 + k * 16, 16)]
            first = v != vp
            last = v != vn
            gp = k.astype(jnp.float32) * 16.0 + io16f
            plsc.store_scatter(cb2, [v], gp, mask=first)
            plsc.store_scatter(cb1, [v], gp + 1.0, mask=last)
            return _
        lax.fori_loop(0, NS // 16, _runs, None)

        def _counts(k, _):
            v = idx_v[pl.ds(IOFF + k * 16, 16)]
            vn = idx_v[pl.ds(IOFF + 1 + k * 16, 16)]
            last = v != vn
            gp = k.astype(jnp.float32) * 16.0 + io16f
            firstpos = plsc.load_gather(cb2, [v])
            plsc.store_scatter(cb1, [v], gp + 1.0 - firstpos, mask=last)
            return _
        lax.fori_loop(0, NS // 16, _counts, None)

        pltpu.sync_copy(cb1, cnt_hbm.at[pl.ds(bl * NPAD, NPAD)])

        def _restore(k, _):
            v = idx_v[pl.ds(IOFF + k * 16, 16)]
            plsc.store_scatter(cb1, [v], z16f)
            plsc.store_scatter(cb2, [v], z16f)
            return _
        lax.fori_loop(0, NS // 16, _restore, None)

        def _adj_set(k, _):
            er = er_v[pl.ds(k * 16, 16)]
            ec = ec_v[pl.ds(k * 16, 16)]
            plsc.store_scatter(adjbuf, [er, ec], one16f)
            return _
        lax.fori_loop(0, ES // 16, _adj_set, None)

        pltpu.sync_copy(adjbuf, adj_hbm.at[pl.ds(bl * NS, NS)])

        def _adj_clr(k, _):
            er = er_v[pl.ds(k * 16, 16)]
            ec = ec_v[pl.ds(k * 16, 16)]
            plsc.store_scatter(adjbuf, [er, ec], z16f)
            return _
        lax.fori_loop(0, ES // 16, _adj_clr, None)
        return _

    lax.fori_loop(0, npw, _node, None)


def _sc_gather_scatter(local, nidx_flat, ee_flat, b0, hb):
    mesh = plsc.VectorSubcoreMesh(
        core_axis_name="c", subcore_axis_name="s",
        num_cores=NC, num_subcores=NSUB)
    fn = functools.partial(
        pl.kernel, functools.partial(_sc_body, b0, hb // NW),
        out_type=[
            jax.ShapeDtypeStruct((hb * NS, DW), jnp.int32),
            jax.ShapeDtypeStruct((hb * NS, NS), jnp.float32),
            jax.ShapeDtypeStruct((hb * NPAD,), jnp.float32),
        ],
        mesh=mesh,
        compiler_params=pltpu.CompilerParams(needs_layout_passes=False),
        scratch_types=[
            pltpu.VMEM((IOFF + NS + 16,), jnp.int32),
            pltpu.VMEM((GCH, DW), jnp.int32),
            pltpu.VMEM((ES,), jnp.int32),
            pltpu.VMEM((ES,), jnp.int32),
            pltpu.VMEM((NS, NS), jnp.float32),
            pltpu.VMEM((NPAD,), jnp.float32),
            pltpu.VMEM((NPAD,), jnp.float32),
            pltpu.SemaphoreType.DMA,
        ],
    )()
    return fn(local, nidx_flat, ee_flat)


def _fdiff_body(c_ref, d_ref, out_ref, acc):
    k = pl.program_id(0)

    @pl.when(k == 0)
    def _():
        acc[...] = jnp.zeros_like(acc)

    acc[...] += lax.dot_general(
        c_ref[...], d_ref[...], (((1,), (0,)), ((), ())),
        preferred_element_type=jnp.float32,
        precision=lax.Precision.HIGHEST)

    @pl.when(k == pl.num_programs(0) - 1)
    def _():
        s = acc[...]
        fd = jnp.sqrt(jnp.sum(s * s, axis=1, keepdims=True)) / NS
        out_ref[...] = jnp.broadcast_to(fd, (out_ref.shape[0], 128))


def _fdiff(cnt, dtab):
    hb = cnt.shape[0]
    return pl.pallas_call(
        _fdiff_body,
        grid=(NPAD // KBLK,),
        in_specs=[
            pl.BlockSpec((hb, KBLK), lambda k: (0, k)),
            pl.BlockSpec((KBLK, D), lambda k: (k, 0)),
        ],
        out_specs=pl.BlockSpec((hb, 128), lambda k: (0, 0)),
        out_shape=jax.ShapeDtypeStruct((hb, 128), jnp.float32),
        scratch_shapes=[pltpu.VMEM((hb, D), jnp.float32)],
    )(cnt, dtab)


def _sdiff_body(teb_ref, adj_ref, u_ref, out_ref):
    tw = teb_ref[...]
    lo = lax.bitcast_convert_type(
        (tw & 0xFFFF).astype(jnp.uint16), jnp.bfloat16)
    hi = lax.bitcast_convert_type(
        (lax.shift_right_logical(tw, 16)).astype(jnp.uint16), jnp.bfloat16)
    dn = (((1,), (1,)), ((), ()))
    sim = (lax.dot_general(lo, lo, dn, preferred_element_type=jnp.float32)
           + lax.dot_general(hi, hi, dn, preferred_element_type=jnp.float32))
    u = u_ref[0]
    g = -jnp.log(-jnp.log(u + 1e-9) + 1e-9)
    z = sim - THRESHOLD + g
    z = z - jnp.max(z, axis=1, keepdims=True)
    e = jnp.exp(z)
    p = e / jnp.sum(e, axis=1, keepdims=True)
    df = adj_ref[...] - p
    sr = jnp.sqrt(jnp.sum(df * df, axis=1, keepdims=True))
    sdiff = jnp.sum(sr) / NS
    out_ref[...] = jnp.full((1, 1, 128), sdiff, jnp.float32)


def _sdiff(teb, adj2, gumbel):
    hb = teb.shape[0] // NS
    return pl.pallas_call(
        _sdiff_body,
        grid=(hb,),
        in_specs=[
            pl.BlockSpec((NS, DW), lambda i: (i, 0)),
            pl.BlockSpec((NS, NS), lambda i: (i, 0)),
            pl.BlockSpec((1, NS, NS), lambda i: (i, 0, 0)),
        ],
        out_specs=pl.BlockSpec((1, 1, 128), lambda i: (i, 0, 0)),
        out_shape=jax.ShapeDtypeStruct((hb, 1, 128), jnp.float32),
    )(teb, adj2, gumbel)


def _final_body(sd_ref, fd_ref, y_ref, score_ref, loss_ref):
    sraw = sd_ref[...] + fd_ref[...]
    mn = jnp.min(sraw)
    mx = jnp.max(sraw)
    sc = (sraw - mn) / (mx - mn)
    y = y_ref[...]
    logp = jnp.maximum(jnp.log(sc), -100.0)
    log1mp = jnp.maximum(jnp.log(1.0 - sc), -100.0)
    bce = -jnp.mean(y * logp + (1.0 - y) * log1mp)
    score_ref[...] = sc
    loss_ref[...] = jnp.full((8, 128), bce / B, jnp.float32)


def _finalize(sdcol, fdcol, ytab):
    return pl.pallas_call(
        _final_body,
        out_shape=[
            jax.ShapeDtypeStruct((B, 128), jnp.float32),
            jax.ShapeDtypeStruct((8, 128), jnp.float32),
        ],
    )(sdcol, fdcol, ytab)


def kernel(feature, text_embedding, virtual, gumbel_noise, train_nodes,
           neighbor_idx, ego_edges, train_label):
    local, dtab = _normalize(feature, text_embedding, virtual)
    nidx_flat = neighbor_idx.reshape(B * NS)
    ee_flat = ego_edges.reshape(B * 2 * ES)
    hb = B // NCHUNK
    sds, fds = [], []
    for c in range(NCHUNK):
        teb, adjf, cnt = _sc_gather_scatter(
            local, nidx_flat, ee_flat, c * hb, hb)
        sd3 = _sdiff(teb, adjf, gumbel_noise[c * hb:(c + 1) * hb])
        fds.append(_fdiff(cnt.reshape(hb, NPAD), dtab))
        sds.append(sd3.reshape(hb, 128))
    sdcol = jnp.concatenate(sds, axis=0)
    fdcol = jnp.concatenate(fds, axis=0)
    ytab = jnp.broadcast_to(
        train_label.astype(jnp.float32)[:, None], (B, 128))
    scoreb, lossb = _finalize(sdcol, fdcol, ytab)
    return scoreb[:, 0], lossb[0, 0]


# speedup vs baseline: 1.1129x; 1.1129x over previous
import functools

import jax
import jax.numpy as jnp
from jax import lax
from jax.experimental import pallas as pl
from jax.experimental.pallas import tpu as pltpu
from jax.experimental.pallas import tpu_sc as plsc

N = 10000
D = 512
B = 256
NS = 256
ES = 2048
THRESHOLD = 0.1
DW = D // 2
NPAD = 10240
KBLK = 2048
RB = 512
NC = 2
NSUB = 16
NW = NC * NSUB
NPW = B // NW
GCH = 64
IOFF = 16
NCHUNK = 1


def _norm_body(f_ref, t_ref, v_ref, local_ref, dtab_ref):
    i = pl.program_id(0)
    f = f_ref[...]
    t = t_ref[...]
    v = v_ref[...]
    eps = 1e-12
    fn = f / jnp.maximum(jnp.sqrt(jnp.sum(f * f, axis=1, keepdims=True)), eps)
    en = t / jnp.maximum(jnp.sqrt(jnp.sum(t * t, axis=1, keepdims=True)), eps)
    lc = en - v
    lcn = lc / jnp.maximum(jnp.sqrt(jnp.sum(lc * lc, axis=1, keepdims=True)), eps)
    lcb = lcn.astype(jnp.bfloat16)
    lo = lax.bitcast_convert_type(lcb[:, :DW], jnp.uint16).astype(jnp.uint32)
    hi = lax.bitcast_convert_type(lcb[:, DW:], jnp.uint16).astype(jnp.uint32)
    local_ref[...] = (lo | (hi << 16)).astype(jnp.int32)
    rid = i * RB + lax.broadcasted_iota(jnp.int32, (RB, 1), 0)
    dtab_ref[...] = jnp.where(rid < N, fn - en, 0.0)


def _normalize(feature, text_embedding, virtual):
    grid = NPAD // RB
    return pl.pallas_call(
        _norm_body,
        grid=(grid,),
        in_specs=[
            pl.BlockSpec((RB, D), lambda i: (i, 0)),
            pl.BlockSpec((RB, D), lambda i: (i, 0)),
            pl.BlockSpec((1, D), lambda i: (0, 0)),
        ],
        out_specs=[
            pl.BlockSpec((RB, DW), lambda i: (i, 0)),
            pl.BlockSpec((RB, D), lambda i: (i, 0)),
        ],
        out_shape=[
            jax.ShapeDtypeStruct((N, DW), jnp.int32),
            jax.ShapeDtypeStruct((NPAD, D), jnp.float32),
        ],
    )(feature, text_embedding, virtual)


def _sc_body(b0, npw, local_hbm, nidx_hbm, ee_hbm,
             teb_hbm, adj_hbm, cnt_hbm,
             idx_v, rows_v, er_v, ec_v, adjbuf, cb1, cb2, sem):
    wid = lax.axis_index("s") * NC + lax.axis_index("c")

    z16f = jnp.zeros((16,), jnp.float32)
    one16f = jnp.ones((16,), jnp.float32)
    io16f = lax.iota(jnp.int32, 16).astype(jnp.float32)

    def _z_adj(r, _):
        for cc in range(NS // 16):
            adjbuf[r, pl.ds(cc * 16, 16)] = z16f
        return _
    lax.fori_loop(0, NS, _z_adj, None)

    def _z_cb(i, _):
        cb1[pl.ds(i * 16, 16)] = z16f
        cb2[pl.ds(i * 16, 16)] = z16f
        return _
    lax.fori_loop(0, NPAD // 16, _z_cb, None)

    def _node(j, _):
        bl = wid * npw + j
        b = b0 + bl

        idx_v[pl.ds(0, 16)] = jnp.full((16,), -1, jnp.int32)
        idx_v[pl.ds(IOFF + NS, 16)] = jnp.full((16,), 0x40000000, jnp.int32)
        pltpu.sync_copy(nidx_hbm.at[pl.ds(b * NS, NS)],
                        idx_v.at[pl.ds(IOFF, NS)])
        pltpu.sync_copy(ee_hbm.at[pl.ds((2 * b) * ES, ES)], er_v)
        pltpu.sync_copy(ee_hbm.at[pl.ds((2 * b + 1) * ES, ES)], ec_v)

        for kk in range(NS // GCH):
            pltpu.async_copy(
                local_hbm.at[idx_v.at[pl.ds(IOFF + kk * GCH, GCH)]],
                rows_v, sem).wait()
            pltpu.sync_copy(rows_v, teb_hbm.at[pl.ds(bl * NS + kk * GCH, GCH)])

        def _runs(k, _):
            v = idx_v[pl.ds(IOFF + k * 16, 16)]
            vp = idx_v[pl.ds(IOFF - 1 + k * 16, 16)]
            vn = idx_v[pl.ds(IOFF + 1 + k * 16, 16)]
            first = v != vp
            last = v != vn
            gp = k.astype(jnp.float32) * 16.0 + io16f
            plsc.store_scatter(cb2, [v], gp, mask=first)
            plsc.store_scatter(cb1, [v], gp + 1.0, mask=last)
            return _
        lax.fori_loop(0, NS // 16, _runs, None)

        def _counts(k, _):
            v = idx_v[pl.ds(IOFF + k * 16, 16)]
            vn = idx_v[pl.ds(IOFF + 1 + k * 16, 16)]
            last = v != vn
            gp = k.astype(jnp.float32) * 16.0 + io16f
            firstpos = plsc.load_gather(cb2, [v])
            plsc.store_scatter(cb1, [v], gp + 1.0 - firstpos, mask=last)
            return _
        lax.fori_loop(0, NS // 16, _counts, None)

        pltpu.sync_copy(cb1, cnt_hbm.at[pl.ds(bl * NPAD, NPAD)])

        def _restore(k, _):
            v = idx_v[pl.ds(IOFF + k * 16, 16)]
            plsc.store_scatter(cb1, [v], z16f)
            plsc.store_scatter(cb2, [v], z16f)
            return _
        lax.fori_loop(0, NS // 16, _restore, None)

        def _adj_set(k, _):
            er = er_v[pl.ds(k * 16, 16)]
            ec = ec_v[pl.ds(k * 16, 16)]
            plsc.store_scatter(adjbuf, [er, ec], one16f)
            return _
        lax.fori_loop(0, ES // 16, _adj_set, None)

        pltpu.sync_copy(adjbuf, adj_hbm.at[pl.ds(bl * NS, NS)])

        def _adj_clr(k, _):
            er = er_v[pl.ds(k * 16, 16)]
            ec = ec_v[pl.ds(k * 16, 16)]
            plsc.store_scatter(adjbuf, [er, ec], z16f)
            return _
        lax.fori_loop(0, ES // 16, _adj_clr, None)
        return _

    lax.fori_loop(0, npw, _node, None)


def _sc_gather_scatter(local, nidx_flat, ee_flat, b0, hb):
    mesh = plsc.VectorSubcoreMesh(
        core_axis_name="c", subcore_axis_name="s",
        num_cores=NC, num_subcores=NSUB)
    fn = functools.partial(
        pl.kernel, functools.partial(_sc_body, b0, hb // NW),
        out_type=[
            jax.ShapeDtypeStruct((hb * NS, DW), jnp.int32),
            jax.ShapeDtypeStruct((hb * NS, NS), jnp.float32),
            jax.ShapeDtypeStruct((hb * NPAD,), jnp.float32),
        ],
        mesh=mesh,
        compiler_params=pltpu.CompilerParams(needs_layout_passes=False),
        scratch_types=[
            pltpu.VMEM((IOFF + NS + 16,), jnp.int32),
            pltpu.VMEM((GCH, DW), jnp.int32),
            pltpu.VMEM((ES,), jnp.int32),
            pltpu.VMEM((ES,), jnp.int32),
            pltpu.VMEM((NS, NS), jnp.float32),
            pltpu.VMEM((NPAD,), jnp.float32),
            pltpu.VMEM((NPAD,), jnp.float32),
            pltpu.SemaphoreType.DMA,
        ],
    )()
    return fn(local, nidx_flat, ee_flat)


def _fdiff_body(c_ref, d_ref, out_ref, acc):
    k = pl.program_id(0)

    @pl.when(k == 0)
    def _():
        acc[...] = jnp.zeros_like(acc)

    acc[...] += lax.dot_general(
        c_ref[...], d_ref[...], (((1,), (0,)), ((), ())),
        preferred_element_type=jnp.float32,
        precision=lax.Precision.HIGHEST)

    @pl.when(k == pl.num_programs(0) - 1)
    def _():
        s = acc[...]
        fd = jnp.sqrt(jnp.sum(s * s, axis=1, keepdims=True)) / NS
        out_ref[...] = jnp.broadcast_to(fd, (out_ref.shape[0], 128))


def _fdiff(cnt, dtab):
    hb = cnt.shape[0]
    return pl.pallas_call(
        _fdiff_body,
        grid=(NPAD // KBLK,),
        in_specs=[
            pl.BlockSpec((hb, KBLK), lambda k: (0, k)),
            pl.BlockSpec((KBLK, D), lambda k: (k, 0)),
        ],
        out_specs=pl.BlockSpec((hb, 128), lambda k: (0, 0)),
        out_shape=jax.ShapeDtypeStruct((hb, 128), jnp.float32),
        scratch_shapes=[pltpu.VMEM((hb, D), jnp.float32)],
    )(cnt, dtab)


def _sdiff_body(teb_ref, adj_ref, u_ref, out_ref):
    tw = teb_ref[...]
    lo = lax.bitcast_convert_type(
        (tw & 0xFFFF).astype(jnp.uint16), jnp.bfloat16)
    hi = lax.bitcast_convert_type(
        (lax.shift_right_logical(tw, 16)).astype(jnp.uint16), jnp.bfloat16)
    dn = (((1,), (1,)), ((), ()))
    sim = (lax.dot_general(lo, lo, dn, preferred_element_type=jnp.float32)
           + lax.dot_general(hi, hi, dn, preferred_element_type=jnp.float32))
    u = u_ref[0]
    g = -jnp.log(-jnp.log(u + 1e-9) + 1e-9)
    z = sim - THRESHOLD + g
    e = jnp.exp(z)
    p = e / jnp.sum(e, axis=1, keepdims=True)
    df = adj_ref[...] - p
    sr = jnp.sqrt(jnp.sum(df * df, axis=1, keepdims=True))
    sdiff = jnp.sum(sr) / NS
    out_ref[...] = jnp.full((1, 1, 128), sdiff, jnp.float32)


def _sdiff(teb, adj2, gumbel):
    hb = teb.shape[0] // NS
    return pl.pallas_call(
        _sdiff_body,
        grid=(hb,),
        in_specs=[
            pl.BlockSpec((NS, DW), lambda i: (i, 0)),
            pl.BlockSpec((NS, NS), lambda i: (i, 0)),
            pl.BlockSpec((1, NS, NS), lambda i: (i, 0, 0)),
        ],
        out_specs=pl.BlockSpec((1, 1, 128), lambda i: (i, 0, 0)),
        out_shape=jax.ShapeDtypeStruct((hb, 1, 128), jnp.float32),
    )(teb, adj2, gumbel)


def _final_body(sd_ref, fd_ref, y_ref, score_ref, loss_ref):
    sraw = sd_ref[...] + fd_ref[...]
    mn = jnp.min(sraw)
    mx = jnp.max(sraw)
    sc = (sraw - mn) / (mx - mn)
    y = y_ref[...]
    logp = jnp.maximum(jnp.log(sc), -100.0)
    log1mp = jnp.maximum(jnp.log(1.0 - sc), -100.0)
    bce = -jnp.mean(y * logp + (1.0 - y) * log1mp)
    score_ref[...] = sc
    loss_ref[...] = jnp.full((8, 128), bce / B, jnp.float32)


def _finalize(sdcol, fdcol, ytab):
    return pl.pallas_call(
        _final_body,
        out_shape=[
            jax.ShapeDtypeStruct((B, 128), jnp.float32),
            jax.ShapeDtypeStruct((8, 128), jnp.float32),
        ],
    )(sdcol, fdcol, ytab)


def kernel(feature, text_embedding, virtual, gumbel_noise, train_nodes,
           neighbor_idx, ego_edges, train_label):
    local, dtab = _normalize(feature, text_embedding, virtual)
    nidx_flat = neighbor_idx.reshape(B * NS)
    ee_flat = ego_edges.reshape(B * 2 * ES)
    hb = B // NCHUNK
    sds, fds = [], []
    for c in range(NCHUNK):
        teb, adjf, cnt = _sc_gather_scatter(
            local, nidx_flat, ee_flat, c * hb, hb)
        sd3 = _sdiff(teb, adjf, gumbel_noise[c * hb:(c + 1) * hb])
        fds.append(_fdiff(cnt.reshape(hb, NPAD), dtab))
        sds.append(sd3.reshape(hb, 128))
    sdcol = jnp.concatenate(sds, axis=0)
    fdcol = jnp.concatenate(fds, axis=0)
    ytab = jnp.broadcast_to(
        train_label.astype(jnp.float32)[:, None], (B, 128))
    scoreb, lossb = _finalize(sdcol, fdcol, ytab)
    return scoreb[:, 0], lossb[0, 0]


# speedup vs baseline: 1.5270x; 1.3721x over previous
import functools

import jax
import jax.numpy as jnp
from jax import lax
from jax.experimental import pallas as pl
from jax.experimental.pallas import tpu as pltpu
from jax.experimental.pallas import tpu_sc as plsc

N = 10000
D = 512
B = 256
NS = 256
ES = 2048
THRESHOLD = 0.1
DW = D // 2
NPAD = 10240
KBLK = 2048
RB = 512
NC = 2
NSUB = 16
NW = NC * NSUB
NPW = B // NW
GCH = 64
IOFF = 16
NCHUNK = 1
MN = 4


def _norm_body(f_ref, t_ref, v_ref, local_ref, dtab_ref):
    i = pl.program_id(0)
    f = f_ref[...]
    t = t_ref[...]
    v = v_ref[...]
    eps = 1e-12
    fn = f / jnp.maximum(jnp.sqrt(jnp.sum(f * f, axis=1, keepdims=True)), eps)
    en = t / jnp.maximum(jnp.sqrt(jnp.sum(t * t, axis=1, keepdims=True)), eps)
    lc = en - v
    lcn = lc / jnp.maximum(jnp.sqrt(jnp.sum(lc * lc, axis=1, keepdims=True)), eps)
    lcb = lcn.astype(jnp.bfloat16)
    lo = lax.bitcast_convert_type(lcb[:, :DW], jnp.uint16).astype(jnp.uint32)
    hi = lax.bitcast_convert_type(lcb[:, DW:], jnp.uint16).astype(jnp.uint32)
    local_ref[...] = (lo | (hi << 16)).astype(jnp.int32)
    rid = i * RB + lax.broadcasted_iota(jnp.int32, (RB, 1), 0)
    dtab_ref[...] = jnp.where(rid < N, fn - en, 0.0)


def _normalize(feature, text_embedding, virtual):
    grid = NPAD // RB
    return pl.pallas_call(
        _norm_body,
        grid=(grid,),
        in_specs=[
            pl.BlockSpec((RB, D), lambda i: (i, 0)),
            pl.BlockSpec((RB, D), lambda i: (i, 0)),
            pl.BlockSpec((1, D), lambda i: (0, 0)),
        ],
        out_specs=[
            pl.BlockSpec((RB, DW), lambda i: (i, 0)),
            pl.BlockSpec((RB, D), lambda i: (i, 0)),
        ],
        out_shape=[
            jax.ShapeDtypeStruct((N, DW), jnp.int32),
            jax.ShapeDtypeStruct((NPAD, D), jnp.float32),
        ],
    )(feature, text_embedding, virtual)


def _sc_body(b0, npw, local_hbm, nidx_hbm, ee_hbm,
             teb_hbm, adj_hbm, cnt_hbm,
             idx_v, rows_v, er_v, ec_v, adjbuf, cb1, cb2, sem):
    wid = lax.axis_index("s") * NC + lax.axis_index("c")

    z16f = jnp.zeros((16,), jnp.float32)
    one16f = jnp.ones((16,), jnp.float32)
    io16f = lax.iota(jnp.int32, 16).astype(jnp.float32)

    def _z_adj(r, _):
        for cc in range(NS // 16):
            adjbuf[r, pl.ds(cc * 16, 16)] = z16f
        return _
    lax.fori_loop(0, NS, _z_adj, None)

    def _z_cb(i, _):
        cb1[pl.ds(i * 16, 16)] = z16f
        cb2[pl.ds(i * 16, 16)] = z16f
        return _
    lax.fori_loop(0, NPAD // 16, _z_cb, None)

    def _node(j, _):
        bl = wid * npw + j
        b = b0 + bl

        idx_v[pl.ds(0, 16)] = jnp.full((16,), -1, jnp.int32)
        idx_v[pl.ds(IOFF + NS, 16)] = jnp.full((16,), 0x40000000, jnp.int32)
        pltpu.sync_copy(nidx_hbm.at[pl.ds(b * NS, NS)],
                        idx_v.at[pl.ds(IOFF, NS)])
        pltpu.sync_copy(ee_hbm.at[pl.ds((2 * b) * ES, ES)], er_v)
        pltpu.sync_copy(ee_hbm.at[pl.ds((2 * b + 1) * ES, ES)], ec_v)

        for kk in range(NS // GCH):
            pltpu.async_copy(
                local_hbm.at[idx_v.at[pl.ds(IOFF + kk * GCH, GCH)]],
                rows_v, sem).wait()
            pltpu.sync_copy(rows_v, teb_hbm.at[pl.ds(bl * NS + kk * GCH, GCH)])

        def _runs(k, _):
            v = idx_v[pl.ds(IOFF + k * 16, 16)]
            vp = idx_v[pl.ds(IOFF - 1 + k * 16, 16)]
            vn = idx_v[pl.ds(IOFF + 1 + k * 16, 16)]
            first = v != vp
            last = v != vn
            gp = k.astype(jnp.float32) * 16.0 + io16f
            plsc.store_scatter(cb2, [v], gp, mask=first)
            plsc.store_scatter(cb1, [v], gp + 1.0, mask=last)
            return _
        lax.fori_loop(0, NS // 16, _runs, None)

        def _counts(k, _):
            v = idx_v[pl.ds(IOFF + k * 16, 16)]
            vn = idx_v[pl.ds(IOFF + 1 + k * 16, 16)]
            last = v != vn
            gp = k.astype(jnp.float32) * 16.0 + io16f
            firstpos = plsc.load_gather(cb2, [v])
            plsc.store_scatter(cb1, [v], gp + 1.0 - firstpos, mask=last)
            return _
        lax.fori_loop(0, NS // 16, _counts, None)

        pltpu.sync_copy(cb1, cnt_hbm.at[pl.ds(bl * NPAD, NPAD)])

        def _restore(k, _):
            v = idx_v[pl.ds(IOFF + k * 16, 16)]
            plsc.store_scatter(cb1, [v], z16f)
            plsc.store_scatter(cb2, [v], z16f)
            return _
        lax.fori_loop(0, NS // 16, _restore, None)

        def _adj_set(k, _):
            er = er_v[pl.ds(k * 16, 16)]
            ec = ec_v[pl.ds(k * 16, 16)]
            plsc.store_scatter(adjbuf, [er, ec], one16f)
            return _
        lax.fori_loop(0, ES // 16, _adj_set, None)

        pltpu.sync_copy(adjbuf, adj_hbm.at[pl.ds(bl * NS, NS)])

        def _adj_clr(k, _):
            er = er_v[pl.ds(k * 16, 16)]
            ec = ec_v[pl.ds(k * 16, 16)]
            plsc.store_scatter(adjbuf, [er, ec], z16f)
            return _
        lax.fori_loop(0, ES // 16, _adj_clr, None)
        return _

    lax.fori_loop(0, npw, _node, None)


def _sc_gather_scatter(local, nidx_flat, ee_flat, b0, hb):
    mesh = plsc.VectorSubcoreMesh(
        core_axis_name="c", subcore_axis_name="s",
        num_cores=NC, num_subcores=NSUB)
    fn = functools.partial(
        pl.kernel, functools.partial(_sc_body, b0, hb // NW),
        out_type=[
            jax.ShapeDtypeStruct((hb * NS, DW), jnp.int32),
            jax.ShapeDtypeStruct((hb * NS, NS), jnp.float32),
            jax.ShapeDtypeStruct((hb * NPAD,), jnp.float32),
        ],
        mesh=mesh,
        compiler_params=pltpu.CompilerParams(needs_layout_passes=False),
        scratch_types=[
            pltpu.VMEM((IOFF + NS + 16,), jnp.int32),
            pltpu.VMEM((GCH, DW), jnp.int32),
            pltpu.VMEM((ES,), jnp.int32),
            pltpu.VMEM((ES,), jnp.int32),
            pltpu.VMEM((NS, NS), jnp.float32),
            pltpu.VMEM((NPAD,), jnp.float32),
            pltpu.VMEM((NPAD,), jnp.float32),
            pltpu.SemaphoreType.DMA,
        ],
    )()
    return fn(local, nidx_flat, ee_flat)


def _fdiff_body(c_ref, d_ref, out_ref, acc):
    k = pl.program_id(0)

    @pl.when(k == 0)
    def _():
        acc[...] = jnp.zeros_like(acc)

    acc[...] += lax.dot_general(
        c_ref[...], d_ref[...], (((1,), (0,)), ((), ())),
        preferred_element_type=jnp.float32,
        precision=lax.Precision.HIGHEST)

    @pl.when(k == pl.num_programs(0) - 1)
    def _():
        s = acc[...]
        fd = jnp.sqrt(jnp.sum(s * s, axis=1, keepdims=True)) / NS
        out_ref[...] = jnp.broadcast_to(fd, (out_ref.shape[0], 128))


def _fdiff(cnt, dtab):
    hb = cnt.shape[0]
    return pl.pallas_call(
        _fdiff_body,
        grid=(NPAD // KBLK,),
        in_specs=[
            pl.BlockSpec((hb, KBLK), lambda k: (0, k)),
            pl.BlockSpec((KBLK, D), lambda k: (k, 0)),
        ],
        out_specs=pl.BlockSpec((hb, 128), lambda k: (0, 0)),
        out_shape=jax.ShapeDtypeStruct((hb, 128), jnp.float32),
        scratch_shapes=[pltpu.VMEM((hb, D), jnp.float32)],
    )(cnt, dtab)


def _sdiff_body(teb_ref, adj_ref, u_ref, out_ref):
    dn = (((1,), (1,)), ((), ()))
    for mn in range(MN):
        tw = teb_ref[pl.ds(mn * NS, NS), :]
        lo = lax.bitcast_convert_type(
            (tw & 0xFFFF).astype(jnp.uint16), jnp.bfloat16)
        hi = lax.bitcast_convert_type(
            (lax.shift_right_logical(tw, 16)).astype(jnp.uint16), jnp.bfloat16)
        sim = (lax.dot_general(lo, lo, dn, preferred_element_type=jnp.float32)
               + lax.dot_general(hi, hi, dn, preferred_element_type=jnp.float32))
        u = u_ref[mn]
        g = -jnp.log(-jnp.log(u + 1e-9) + 1e-9)
        z = sim - THRESHOLD + g
        e = jnp.exp(z)
        p = e / jnp.sum(e, axis=1, keepdims=True)
        df = adj_ref[pl.ds(mn * NS, NS), :] - p
        sr = jnp.sqrt(jnp.sum(df * df, axis=1, keepdims=True))
        sdiff = jnp.sum(sr) / NS
        out_ref[pl.ds(mn, 1)] = jnp.full((1, 1, 128), sdiff, jnp.float32)


def _sdiff(teb, adj2, gumbel):
    hb = teb.shape[0] // NS
    return pl.pallas_call(
        _sdiff_body,
        grid=(hb // MN,),
        in_specs=[
            pl.BlockSpec((MN * NS, DW), lambda i: (i, 0)),
            pl.BlockSpec((MN * NS, NS), lambda i: (i, 0)),
            pl.BlockSpec((MN, NS, NS), lambda i: (i, 0, 0)),
        ],
        out_specs=pl.BlockSpec((MN, 1, 128), lambda i: (i, 0, 0)),
        out_shape=jax.ShapeDtypeStruct((hb, 1, 128), jnp.float32),
    )(teb, adj2, gumbel)


def _final_body(sd_ref, fd_ref, y_ref, score_ref, loss_ref):
    sraw = sd_ref[...] + fd_ref[...]
    mn = jnp.min(sraw)
    mx = jnp.max(sraw)
    sc = (sraw - mn) / (mx - mn)
    y = y_ref[...]
    logp = jnp.maximum(jnp.log(sc), -100.0)
    log1mp = jnp.maximum(jnp.log(1.0 - sc), -100.0)
    bce = -jnp.mean(y * logp + (1.0 - y) * log1mp)
    score_ref[...] = sc
    loss_ref[...] = jnp.full((8, 128), bce / B, jnp.float32)


def _finalize(sdcol, fdcol, ytab):
    return pl.pallas_call(
        _final_body,
        out_shape=[
            jax.ShapeDtypeStruct((B, 128), jnp.float32),
            jax.ShapeDtypeStruct((8, 128), jnp.float32),
        ],
    )(sdcol, fdcol, ytab)


def kernel(feature, text_embedding, virtual, gumbel_noise, train_nodes,
           neighbor_idx, ego_edges, train_label):
    local, dtab = _normalize(feature, text_embedding, virtual)
    nidx_flat = neighbor_idx.reshape(B * NS)
    ee_flat = ego_edges.reshape(B * 2 * ES)
    hb = B // NCHUNK
    sds, fds = [], []
    for c in range(NCHUNK):
        teb, adjf, cnt = _sc_gather_scatter(
            local, nidx_flat, ee_flat, c * hb, hb)
        sd3 = _sdiff(teb, adjf, gumbel_noise[c * hb:(c + 1) * hb])
        fds.append(_fdiff(cnt.reshape(hb, NPAD), dtab))
        sds.append(sd3.reshape(hb, 128))
    sdcol = jnp.concatenate(sds, axis=0)
    fdcol = jnp.concatenate(fds, axis=0)
    ytab = jnp.broadcast_to(
        train_label.astype(jnp.float32)[:, None], (B, 128))
    scoreb, lossb = _finalize(sdcol, fdcol, ytab)
    return scoreb[:, 0], lossb[0, 0]


# speedup vs baseline: 1.6151x; 1.0577x over previous
import functools

import jax
import jax.numpy as jnp
from jax import lax
from jax.experimental import pallas as pl
from jax.experimental.pallas import tpu as pltpu
from jax.experimental.pallas import tpu_sc as plsc

N = 10000
D = 512
B = 256
NS = 256
ES = 2048
THRESHOLD = 0.1
DW = D // 2
NPAD = 10240
KBLK = 2048
RB = 512
NC = 2
NSUB = 16
NW = NC * NSUB
NPW = B // NW
GCH = 64
IOFF = 16
NCHUNK = 1
MN = 8


def _norm_body(f_ref, t_ref, v_ref, local_ref, dtab_ref):
    i = pl.program_id(0)
    f = f_ref[...]
    t = t_ref[...]
    v = v_ref[...]
    eps = 1e-12
    fn = f / jnp.maximum(jnp.sqrt(jnp.sum(f * f, axis=1, keepdims=True)), eps)
    en = t / jnp.maximum(jnp.sqrt(jnp.sum(t * t, axis=1, keepdims=True)), eps)
    lc = en - v
    lcn = lc / jnp.maximum(jnp.sqrt(jnp.sum(lc * lc, axis=1, keepdims=True)), eps)
    lcb = lcn.astype(jnp.bfloat16)
    lo = lax.bitcast_convert_type(lcb[:, :DW], jnp.uint16).astype(jnp.uint32)
    hi = lax.bitcast_convert_type(lcb[:, DW:], jnp.uint16).astype(jnp.uint32)
    local_ref[...] = (lo | (hi << 16)).astype(jnp.int32)
    rid = i * RB + lax.broadcasted_iota(jnp.int32, (RB, 1), 0)
    dtab_ref[...] = jnp.where(rid < N, fn - en, 0.0)


def _normalize(feature, text_embedding, virtual):
    grid = NPAD // RB
    return pl.pallas_call(
        _norm_body,
        grid=(grid,),
        in_specs=[
            pl.BlockSpec((RB, D), lambda i: (i, 0)),
            pl.BlockSpec((RB, D), lambda i: (i, 0)),
            pl.BlockSpec((1, D), lambda i: (0, 0)),
        ],
        out_specs=[
            pl.BlockSpec((RB, DW), lambda i: (i, 0)),
            pl.BlockSpec((RB, D), lambda i: (i, 0)),
        ],
        out_shape=[
            jax.ShapeDtypeStruct((N, DW), jnp.int32),
            jax.ShapeDtypeStruct((NPAD, D), jnp.float32),
        ],
    )(feature, text_embedding, virtual)


def _sc_body(b0, npw, local_hbm, nidx_hbm, ee_hbm,
             teb_hbm, adj_hbm, cnt_hbm,
             idx_v, rows_v, er_v, ec_v, adjbuf, cb1, cb2, sem):
    wid = lax.axis_index("s") * NC + lax.axis_index("c")

    z16f = jnp.zeros((16,), jnp.float32)
    one16f = jnp.ones((16,), jnp.float32)
    io16f = lax.iota(jnp.int32, 16).astype(jnp.float32)

    def _z_adj(r, _):
        for cc in range(NS // 16):
            adjbuf[r, pl.ds(cc * 16, 16)] = z16f
        return _
    lax.fori_loop(0, NS, _z_adj, None)

    def _z_cb(i, _):
        cb1[pl.ds(i * 16, 16)] = z16f
        cb2[pl.ds(i * 16, 16)] = z16f
        return _
    lax.fori_loop(0, NPAD // 16, _z_cb, None)

    def _node(j, _):
        bl = wid * npw + j
        b = b0 + bl

        idx_v[pl.ds(0, 16)] = jnp.full((16,), -1, jnp.int32)
        idx_v[pl.ds(IOFF + NS, 16)] = jnp.full((16,), 0x40000000, jnp.int32)
        pltpu.sync_copy(nidx_hbm.at[pl.ds(b * NS, NS)],
                        idx_v.at[pl.ds(IOFF, NS)])
        pltpu.sync_copy(ee_hbm.at[pl.ds((2 * b) * ES, ES)], er_v)
        pltpu.sync_copy(ee_hbm.at[pl.ds((2 * b + 1) * ES, ES)], ec_v)

        for kk in range(NS // GCH):
            pltpu.async_copy(
                local_hbm.at[idx_v.at[pl.ds(IOFF + kk * GCH, GCH)]],
                rows_v, sem).wait()
            pltpu.sync_copy(rows_v, teb_hbm.at[pl.ds(bl * NS + kk * GCH, GCH)])

        def _runs(k, _):
            v = idx_v[pl.ds(IOFF + k * 16, 16)]
            vp = idx_v[pl.ds(IOFF - 1 + k * 16, 16)]
            vn = idx_v[pl.ds(IOFF + 1 + k * 16, 16)]
            first = v != vp
            last = v != vn
            gp = k.astype(jnp.float32) * 16.0 + io16f
            plsc.store_scatter(cb2, [v], gp, mask=first)
            plsc.store_scatter(cb1, [v], gp + 1.0, mask=last)
            return _
        lax.fori_loop(0, NS // 16, _runs, None)

        def _counts(k, _):
            v = idx_v[pl.ds(IOFF + k * 16, 16)]
            vn = idx_v[pl.ds(IOFF + 1 + k * 16, 16)]
            last = v != vn
            gp = k.astype(jnp.float32) * 16.0 + io16f
            firstpos = plsc.load_gather(cb2, [v])
            plsc.store_scatter(cb1, [v], gp + 1.0 - firstpos, mask=last)
            return _
        lax.fori_loop(0, NS // 16, _counts, None)

        pltpu.sync_copy(cb1, cnt_hbm.at[pl.ds(bl * NPAD, NPAD)])

        def _restore(k, _):
            v = idx_v[pl.ds(IOFF + k * 16, 16)]
            plsc.store_scatter(cb1, [v], z16f)
            plsc.store_scatter(cb2, [v], z16f)
            return _
        lax.fori_loop(0, NS // 16, _restore, None)

        def _adj_set(k, _):
            er = er_v[pl.ds(k * 16, 16)]
            ec = ec_v[pl.ds(k * 16, 16)]
            plsc.store_scatter(adjbuf, [er, ec], one16f)
            return _
        lax.fori_loop(0, ES // 16, _adj_set, None)

        pltpu.sync_copy(adjbuf, adj_hbm.at[pl.ds(bl * NS, NS)])

        def _adj_clr(k, _):
            er = er_v[pl.ds(k * 16, 16)]
            ec = ec_v[pl.ds(k * 16, 16)]
            plsc.store_scatter(adjbuf, [er, ec], z16f)
            return _
        lax.fori_loop(0, ES // 16, _adj_clr, None)
        return _

    lax.fori_loop(0, npw, _node, None)


def _sc_gather_scatter(local, nidx_flat, ee_flat, b0, hb):
    mesh = plsc.VectorSubcoreMesh(
        core_axis_name="c", subcore_axis_name="s",
        num_cores=NC, num_subcores=NSUB)
    fn = functools.partial(
        pl.kernel, functools.partial(_sc_body, b0, hb // NW),
        out_type=[
            jax.ShapeDtypeStruct((hb * NS, DW), jnp.int32),
            jax.ShapeDtypeStruct((hb * NS, NS), jnp.float32),
            jax.ShapeDtypeStruct((hb * NPAD,), jnp.float32),
        ],
        mesh=mesh,
        compiler_params=pltpu.CompilerParams(needs_layout_passes=False),
        scratch_types=[
            pltpu.VMEM((IOFF + NS + 16,), jnp.int32),
            pltpu.VMEM((GCH, DW), jnp.int32),
            pltpu.VMEM((ES,), jnp.int32),
            pltpu.VMEM((ES,), jnp.int32),
            pltpu.VMEM((NS, NS), jnp.float32),
            pltpu.VMEM((NPAD,), jnp.float32),
            pltpu.VMEM((NPAD,), jnp.float32),
            pltpu.SemaphoreType.DMA,
        ],
    )()
    return fn(local, nidx_flat, ee_flat)


def _fdiff_body(c_ref, d_ref, out_ref, acc):
    k = pl.program_id(0)

    @pl.when(k == 0)
    def _():
        acc[...] = jnp.zeros_like(acc)

    acc[...] += lax.dot_general(
        c_ref[...], d_ref[...], (((1,), (0,)), ((), ())),
        preferred_element_type=jnp.float32,
        precision=lax.Precision.HIGHEST)

    @pl.when(k == pl.num_programs(0) - 1)
    def _():
        s = acc[...]
        fd = jnp.sqrt(jnp.sum(s * s, axis=1, keepdims=True)) / NS
        out_ref[...] = jnp.broadcast_to(fd, (out_ref.shape[0], 128))


def _fdiff(cnt, dtab):
    hb = cnt.shape[0]
    return pl.pallas_call(
        _fdiff_body,
        grid=(NPAD // KBLK,),
        in_specs=[
            pl.BlockSpec((hb, KBLK), lambda k: (0, k)),
            pl.BlockSpec((KBLK, D), lambda k: (k, 0)),
        ],
        out_specs=pl.BlockSpec((hb, 128), lambda k: (0, 0)),
        out_shape=jax.ShapeDtypeStruct((hb, 128), jnp.float32),
        scratch_shapes=[pltpu.VMEM((hb, D), jnp.float32)],
    )(cnt, dtab)


def _sdiff_body(teb_ref, adj_ref, u_ref, out_ref):
    dn = (((1,), (1,)), ((), ()))
    for mn in range(MN):
        tw = teb_ref[pl.ds(mn * NS, NS), :]
        lo = lax.bitcast_convert_type(
            (tw & 0xFFFF).astype(jnp.uint16), jnp.bfloat16)
        hi = lax.bitcast_convert_type(
            (lax.shift_right_logical(tw, 16)).astype(jnp.uint16), jnp.bfloat16)
        sim = (lax.dot_general(lo, lo, dn, preferred_element_type=jnp.float32)
               + lax.dot_general(hi, hi, dn, preferred_element_type=jnp.float32))
        u = u_ref[mn]
        g = -jnp.log(-jnp.log(u + 1e-9) + 1e-9)
        z = sim - THRESHOLD + g
        e = jnp.exp(z)
        p = e / jnp.sum(e, axis=1, keepdims=True)
        df = adj_ref[pl.ds(mn * NS, NS), :] - p
        sr = jnp.sqrt(jnp.sum(df * df, axis=1, keepdims=True))
        sdiff = jnp.sum(sr) / NS
        out_ref[pl.ds(mn, 1)] = jnp.full((1, 1, 128), sdiff, jnp.float32)


def _sdiff(teb, adj2, gumbel):
    hb = teb.shape[0] // NS
    return pl.pallas_call(
        _sdiff_body,
        grid=(hb // MN,),
        in_specs=[
            pl.BlockSpec((MN * NS, DW), lambda i: (i, 0)),
            pl.BlockSpec((MN * NS, NS), lambda i: (i, 0)),
            pl.BlockSpec((MN, NS, NS), lambda i: (i, 0, 0)),
        ],
        out_specs=pl.BlockSpec((MN, 1, 128), lambda i: (i, 0, 0)),
        out_shape=jax.ShapeDtypeStruct((hb, 1, 128), jnp.float32),
    )(teb, adj2, gumbel)


def _final_body(sd_ref, fd_ref, y_ref, score_ref, loss_ref):
    sraw = sd_ref[...] + fd_ref[...]
    mn = jnp.min(sraw)
    mx = jnp.max(sraw)
    sc = (sraw - mn) / (mx - mn)
    y = y_ref[...]
    logp = jnp.maximum(jnp.log(sc), -100.0)
    log1mp = jnp.maximum(jnp.log(1.0 - sc), -100.0)
    bce = -jnp.mean(y * logp + (1.0 - y) * log1mp)
    score_ref[...] = sc
    loss_ref[...] = jnp.full((8, 128), bce / B, jnp.float32)


def _finalize(sdcol, fdcol, ytab):
    return pl.pallas_call(
        _final_body,
        out_shape=[
            jax.ShapeDtypeStruct((B, 128), jnp.float32),
            jax.ShapeDtypeStruct((8, 128), jnp.float32),
        ],
    )(sdcol, fdcol, ytab)


def kernel(feature, text_embedding, virtual, gumbel_noise, train_nodes,
           neighbor_idx, ego_edges, train_label):
    local, dtab = _normalize(feature, text_embedding, virtual)
    nidx_flat = neighbor_idx.reshape(B * NS)
    ee_flat = ego_edges.reshape(B * 2 * ES)
    hb = B // NCHUNK
    sds, fds = [], []
    for c in range(NCHUNK):
        teb, adjf, cnt = _sc_gather_scatter(
            local, nidx_flat, ee_flat, c * hb, hb)
        sd3 = _sdiff(teb, adjf, gumbel_noise[c * hb:(c + 1) * hb])
        fds.append(_fdiff(cnt.reshape(hb, NPAD), dtab))
        sds.append(sd3.reshape(hb, 128))
    sdcol = jnp.concatenate(sds, axis=0)
    fdcol = jnp.concatenate(fds, axis=0)
    ytab = jnp.broadcast_to(
        train_label.astype(jnp.float32)[:, None], (B, 128))
    scoreb, lossb = _finalize(sdcol, fdcol, ytab)
    return scoreb[:, 0], lossb[0, 0]


# speedup vs baseline: 1.6560x; 1.0253x over previous
import functools

import jax
import jax.numpy as jnp
from jax import lax
from jax.experimental import pallas as pl
from jax.experimental.pallas import tpu as pltpu
from jax.experimental.pallas import tpu_sc as plsc

N = 10000
D = 512
B = 256
NS = 256
ES = 2048
THRESHOLD = 0.1
DW = D // 2
NPAD = 10240
KBLK = 2048
RB = 512
NC = 2
NSUB = 16
NW = NC * NSUB
NPW = B // NW
GCH = 64
IOFF = 16
NCHUNK = 1
MN = 16


def _norm_body(f_ref, t_ref, v_ref, local_ref, dtab_ref):
    i = pl.program_id(0)
    f = f_ref[...]
    t = t_ref[...]
    v = v_ref[...]
    eps = 1e-12
    fn = f / jnp.maximum(jnp.sqrt(jnp.sum(f * f, axis=1, keepdims=True)), eps)
    en = t / jnp.maximum(jnp.sqrt(jnp.sum(t * t, axis=1, keepdims=True)), eps)
    lc = en - v
    lcn = lc / jnp.maximum(jnp.sqrt(jnp.sum(lc * lc, axis=1, keepdims=True)), eps)
    lcb = lcn.astype(jnp.bfloat16)
    lo = lax.bitcast_convert_type(lcb[:, :DW], jnp.uint16).astype(jnp.uint32)
    hi = lax.bitcast_convert_type(lcb[:, DW:], jnp.uint16).astype(jnp.uint32)
    local_ref[...] = (lo | (hi << 16)).astype(jnp.int32)
    rid = i * RB + lax.broadcasted_iota(jnp.int32, (RB, 1), 0)
    dtab_ref[...] = jnp.where(rid < N, fn - en, 0.0)


def _normalize(feature, text_embedding, virtual):
    grid = NPAD // RB
    return pl.pallas_call(
        _norm_body,
        grid=(grid,),
        in_specs=[
            pl.BlockSpec((RB, D), lambda i: (i, 0)),
            pl.BlockSpec((RB, D), lambda i: (i, 0)),
            pl.BlockSpec((1, D), lambda i: (0, 0)),
        ],
        out_specs=[
            pl.BlockSpec((RB, DW), lambda i: (i, 0)),
            pl.BlockSpec((RB, D), lambda i: (i, 0)),
        ],
        out_shape=[
            jax.ShapeDtypeStruct((N, DW), jnp.int32),
            jax.ShapeDtypeStruct((NPAD, D), jnp.float32),
        ],
    )(feature, text_embedding, virtual)


def _sc_body(b0, npw, local_hbm, nidx_hbm, ee_hbm,
             teb_hbm, adj_hbm, cnt_hbm,
             idx_v, rows_v, er_v, ec_v, adjbuf, cb1, cb2, sem):
    wid = lax.axis_index("s") * NC + lax.axis_index("c")

    z16f = jnp.zeros((16,), jnp.float32)
    one16f = jnp.ones((16,), jnp.float32)
    io16f = lax.iota(jnp.int32, 16).astype(jnp.float32)

    def _z_adj(r, _):
        for cc in range(NS // 16):
            adjbuf[r, pl.ds(cc * 16, 16)] = z16f
        return _
    lax.fori_loop(0, NS, _z_adj, None)

    def _z_cb(i, _):
        cb1[pl.ds(i * 16, 16)] = z16f
        cb2[pl.ds(i * 16, 16)] = z16f
        return _
    lax.fori_loop(0, NPAD // 16, _z_cb, None)

    def _node(j, _):
        bl = wid * npw + j
        b = b0 + bl

        idx_v[pl.ds(0, 16)] = jnp.full((16,), -1, jnp.int32)
        idx_v[pl.ds(IOFF + NS, 16)] = jnp.full((16,), 0x40000000, jnp.int32)
        pltpu.sync_copy(nidx_hbm.at[pl.ds(b * NS, NS)],
                        idx_v.at[pl.ds(IOFF, NS)])
        pltpu.sync_copy(ee_hbm.at[pl.ds((2 * b) * ES, ES)], er_v)
        pltpu.sync_copy(ee_hbm.at[pl.ds((2 * b + 1) * ES, ES)], ec_v)

        for kk in range(NS // GCH):
            pltpu.async_copy(
                local_hbm.at[idx_v.at[pl.ds(IOFF + kk * GCH, GCH)]],
                rows_v, sem).wait()
            pltpu.sync_copy(rows_v, teb_hbm.at[pl.ds(bl * NS + kk * GCH, GCH)])

        def _runs(k, _):
            v = idx_v[pl.ds(IOFF + k * 16, 16)]
            vp = idx_v[pl.ds(IOFF - 1 + k * 16, 16)]
            vn = idx_v[pl.ds(IOFF + 1 + k * 16, 16)]
            first = v != vp
            last = v != vn
            gp = k.astype(jnp.float32) * 16.0 + io16f
            plsc.store_scatter(cb2, [v], gp, mask=first)
            plsc.store_scatter(cb1, [v], gp + 1.0, mask=last)
            return _
        lax.fori_loop(0, NS // 16, _runs, None)

        def _counts(k, _):
            v = idx_v[pl.ds(IOFF + k * 16, 16)]
            vn = idx_v[pl.ds(IOFF + 1 + k * 16, 16)]
            last = v != vn
            gp = k.astype(jnp.float32) * 16.0 + io16f
            firstpos = plsc.load_gather(cb2, [v])
            plsc.store_scatter(cb1, [v], gp + 1.0 - firstpos, mask=last)
            return _
        lax.fori_loop(0, NS // 16, _counts, None)

        pltpu.sync_copy(cb1, cnt_hbm.at[pl.ds(bl * NPAD, NPAD)])

        def _restore(k, _):
            v = idx_v[pl.ds(IOFF + k * 16, 16)]
            plsc.store_scatter(cb1, [v], z16f)
            plsc.store_scatter(cb2, [v], z16f)
            return _
        lax.fori_loop(0, NS // 16, _restore, None)

        def _adj_set(k, _):
            er = er_v[pl.ds(k * 16, 16)]
            ec = ec_v[pl.ds(k * 16, 16)]
            plsc.store_scatter(adjbuf, [er, ec], one16f)
            return _
        lax.fori_loop(0, ES // 16, _adj_set, None)

        pltpu.sync_copy(adjbuf, adj_hbm.at[pl.ds(bl * NS, NS)])

        def _adj_clr(k, _):
            er = er_v[pl.ds(k * 16, 16)]
            ec = ec_v[pl.ds(k * 16, 16)]
            plsc.store_scatter(adjbuf, [er, ec], z16f)
            return _
        lax.fori_loop(0, ES // 16, _adj_clr, None)
        return _

    lax.fori_loop(0, npw, _node, None)


def _sc_gather_scatter(local, nidx_flat, ee_flat, b0, hb):
    mesh = plsc.VectorSubcoreMesh(
        core_axis_name="c", subcore_axis_name="s",
        num_cores=NC, num_subcores=NSUB)
    fn = functools.partial(
        pl.kernel, functools.partial(_sc_body, b0, hb // NW),
        out_type=[
            jax.ShapeDtypeStruct((hb * NS, DW), jnp.int32),
            jax.ShapeDtypeStruct((hb * NS, NS), jnp.float32),
            jax.ShapeDtypeStruct((hb * NPAD,), jnp.float32),
        ],
        mesh=mesh,
        compiler_params=pltpu.CompilerParams(needs_layout_passes=False),
        scratch_types=[
            pltpu.VMEM((IOFF + NS + 16,), jnp.int32),
            pltpu.VMEM((GCH, DW), jnp.int32),
            pltpu.VMEM((ES,), jnp.int32),
            pltpu.VMEM((ES,), jnp.int32),
            pltpu.VMEM((NS, NS), jnp.float32),
            pltpu.VMEM((NPAD,), jnp.float32),
            pltpu.VMEM((NPAD,), jnp.float32),
            pltpu.SemaphoreType.DMA,
        ],
    )()
    return fn(local, nidx_flat, ee_flat)


def _fdiff_body(c_ref, d_ref, out_ref, acc):
    k = pl.program_id(0)

    @pl.when(k == 0)
    def _():
        acc[...] = jnp.zeros_like(acc)

    acc[...] += lax.dot_general(
        c_ref[...], d_ref[...], (((1,), (0,)), ((), ())),
        preferred_element_type=jnp.float32,
        precision=lax.Precision.HIGHEST)

    @pl.when(k == pl.num_programs(0) - 1)
    def _():
        s = acc[...]
        fd = jnp.sqrt(jnp.sum(s * s, axis=1, keepdims=True)) / NS
        out_ref[...] = jnp.broadcast_to(fd, (out_ref.shape[0], 128))


def _fdiff(cnt, dtab):
    hb = cnt.shape[0]
    return pl.pallas_call(
        _fdiff_body,
        grid=(NPAD // KBLK,),
        in_specs=[
            pl.BlockSpec((hb, KBLK), lambda k: (0, k)),
            pl.BlockSpec((KBLK, D), lambda k: (k, 0)),
        ],
        out_specs=pl.BlockSpec((hb, 128), lambda k: (0, 0)),
        out_shape=jax.ShapeDtypeStruct((hb, 128), jnp.float32),
        scratch_shapes=[pltpu.VMEM((hb, D), jnp.float32)],
    )(cnt, dtab)


def _sdiff_body(teb_ref, adj_ref, u_ref, out_ref):
    dn = (((1,), (1,)), ((), ()))
    for mn in range(MN):
        tw = teb_ref[pl.ds(mn * NS, NS), :]
        lo = lax.bitcast_convert_type(
            (tw & 0xFFFF).astype(jnp.uint16), jnp.bfloat16)
        hi = lax.bitcast_convert_type(
            (lax.shift_right_logical(tw, 16)).astype(jnp.uint16), jnp.bfloat16)
        sim = (lax.dot_general(lo, lo, dn, preferred_element_type=jnp.float32)
               + lax.dot_general(hi, hi, dn, preferred_element_type=jnp.float32))
        u = u_ref[mn]
        g = -jnp.log(-jnp.log(u + 1e-9) + 1e-9)
        z = sim - THRESHOLD + g
        e = jnp.exp(z)
        p = e / jnp.sum(e, axis=1, keepdims=True)
        df = adj_ref[pl.ds(mn * NS, NS), :] - p
        sr = jnp.sqrt(jnp.sum(df * df, axis=1, keepdims=True))
        sdiff = jnp.sum(sr) / NS
        out_ref[pl.ds(mn, 1)] = jnp.full((1, 1, 128), sdiff, jnp.float32)


def _sdiff(teb, adj2, gumbel):
    hb = teb.shape[0] // NS
    return pl.pallas_call(
        _sdiff_body,
        grid=(hb // MN,),
        in_specs=[
            pl.BlockSpec((MN * NS, DW), lambda i: (i, 0)),
            pl.BlockSpec((MN * NS, NS), lambda i: (i, 0)),
            pl.BlockSpec((MN, NS, NS), lambda i: (i, 0, 0)),
        ],
        out_specs=pl.BlockSpec((MN, 1, 128), lambda i: (i, 0, 0)),
        out_shape=jax.ShapeDtypeStruct((hb, 1, 128), jnp.float32),
    )(teb, adj2, gumbel)


def _final_body(sd_ref, fd_ref, y_ref, score_ref, loss_ref):
    sraw = sd_ref[...] + fd_ref[...]
    mn = jnp.min(sraw)
    mx = jnp.max(sraw)
    sc = (sraw - mn) / (mx - mn)
    y = y_ref[...]
    logp = jnp.maximum(jnp.log(sc), -100.0)
    log1mp = jnp.maximum(jnp.log(1.0 - sc), -100.0)
    bce = -jnp.mean(y * logp + (1.0 - y) * log1mp)
    score_ref[...] = sc
    loss_ref[...] = jnp.full((8, 128), bce / B, jnp.float32)


def _finalize(sdcol, fdcol, ytab):
    return pl.pallas_call(
        _final_body,
        out_shape=[
            jax.ShapeDtypeStruct((B, 128), jnp.float32),
            jax.ShapeDtypeStruct((8, 128), jnp.float32),
        ],
    )(sdcol, fdcol, ytab)


def kernel(feature, text_embedding, virtual, gumbel_noise, train_nodes,
           neighbor_idx, ego_edges, train_label):
    local, dtab = _normalize(feature, text_embedding, virtual)
    nidx_flat = neighbor_idx.reshape(B * NS)
    ee_flat = ego_edges.reshape(B * 2 * ES)
    hb = B // NCHUNK
    sds, fds = [], []
    for c in range(NCHUNK):
        teb, adjf, cnt = _sc_gather_scatter(
            local, nidx_flat, ee_flat, c * hb, hb)
        sd3 = _sdiff(teb, adjf, gumbel_noise[c * hb:(c + 1) * hb])
        fds.append(_fdiff(cnt.reshape(hb, NPAD), dtab))
        sds.append(sd3.reshape(hb, 128))
    sdcol = jnp.concatenate(sds, axis=0)
    fdcol = jnp.concatenate(fds, axis=0)
    ytab = jnp.broadcast_to(
        train_label.astype(jnp.float32)[:, None], (B, 128))
    scoreb, lossb = _finalize(sdcol, fdcol, ytab)
    return scoreb[:, 0], lossb[0, 0]


# speedup vs baseline: 1.7852x; 1.0780x over previous
import functools

import jax
import jax.numpy as jnp
from jax import lax
from jax.experimental import pallas as pl
from jax.experimental.pallas import tpu as pltpu
from jax.experimental.pallas import tpu_sc as plsc

N = 10000
D = 512
B = 256
NS = 256
ES = 2048
THRESHOLD = 0.1
DW = D // 2
NPAD = 10240
KBLK = 2048
RB = 512
NC = 2
NSUB = 16
NW = NC * NSUB
NPW = B // NW
GCH = 64
IOFF = 16
NCHUNK = 1
MN = 16


def _norm_body(f_ref, t_ref, v_ref, local_ref, dtab_ref):
    i = pl.program_id(0)
    f = f_ref[...]
    t = t_ref[...]
    v = v_ref[...]
    eps = 1e-12
    fn = f / jnp.maximum(jnp.sqrt(jnp.sum(f * f, axis=1, keepdims=True)), eps)
    en = t / jnp.maximum(jnp.sqrt(jnp.sum(t * t, axis=1, keepdims=True)), eps)
    lc = en - v
    lcn = lc / jnp.maximum(jnp.sqrt(jnp.sum(lc * lc, axis=1, keepdims=True)), eps)
    lcb = lcn.astype(jnp.bfloat16)
    lo = lax.bitcast_convert_type(lcb[:, :DW], jnp.uint16).astype(jnp.uint32)
    hi = lax.bitcast_convert_type(lcb[:, DW:], jnp.uint16).astype(jnp.uint32)
    local_ref[...] = (lo | (hi << 16)).astype(jnp.int32)
    rid = i * RB + lax.broadcasted_iota(jnp.int32, (RB, 1), 0)
    dtab_ref[...] = jnp.where(rid < N, fn - en, 0.0)


def _normalize(feature, text_embedding, virtual):
    grid = NPAD // RB
    return pl.pallas_call(
        _norm_body,
        grid=(grid,),
        in_specs=[
            pl.BlockSpec((RB, D), lambda i: (i, 0)),
            pl.BlockSpec((RB, D), lambda i: (i, 0)),
            pl.BlockSpec((1, D), lambda i: (0, 0)),
        ],
        out_specs=[
            pl.BlockSpec((RB, DW), lambda i: (i, 0)),
            pl.BlockSpec((RB, D), lambda i: (i, 0)),
        ],
        out_shape=[
            jax.ShapeDtypeStruct((N, DW), jnp.int32),
            jax.ShapeDtypeStruct((NPAD, D), jnp.float32),
        ],
    )(feature, text_embedding, virtual)


def _sc_body(b0, npw, local_hbm, nidx_hbm, ee_hbm,
             teb_hbm, adj_hbm, cnt_hbm,
             idx_v, rows_v, rows_v2, er_v, ec_v, adjbuf, cb1, cb2,
             sem, sem2):
    wid = lax.axis_index("s") * NC + lax.axis_index("c")

    z16f = jnp.zeros((16,), jnp.float32)
    one16f = jnp.ones((16,), jnp.float32)
    io16f = lax.iota(jnp.int32, 16).astype(jnp.float32)

    def _z_adj(r, _):
        for cc in range(NS // 16):
            adjbuf[r, pl.ds(cc * 16, 16)] = z16f
        return _
    lax.fori_loop(0, NS, _z_adj, None)

    def _z_cb(i, _):
        cb1[pl.ds(i * 16, 16)] = z16f
        cb2[pl.ds(i * 16, 16)] = z16f
        return _
    lax.fori_loop(0, NPAD // 16, _z_cb, None)

    def _node(j, _):
        bl = wid * npw + j
        b = b0 + bl

        idx_v[pl.ds(0, 16)] = jnp.full((16,), -1, jnp.int32)
        idx_v[pl.ds(IOFF + NS, 16)] = jnp.full((16,), 0x40000000, jnp.int32)
        pltpu.sync_copy(nidx_hbm.at[pl.ds(b * NS, NS)],
                        idx_v.at[pl.ds(IOFF, NS)])
        pltpu.sync_copy(ee_hbm.at[pl.ds((2 * b) * ES, ES)], er_v)
        pltpu.sync_copy(ee_hbm.at[pl.ds((2 * b + 1) * ES, ES)], ec_v)

        def _adj_set(k, _):
            er = er_v[pl.ds(k * 16, 16)]
            ec = ec_v[pl.ds(k * 16, 16)]
            plsc.store_scatter(adjbuf, [er, ec], one16f)
            return _
        lax.fori_loop(0, ES // 16, _adj_set, None)
        adj_dma = pltpu.async_copy(
            adjbuf, adj_hbm.at[pl.ds(bl * NS, NS)], sem2)

        bufs = [rows_v, rows_v2]
        g = pltpu.async_copy(
            local_hbm.at[idx_v.at[pl.ds(IOFF, GCH)]], bufs[0], sem)
        for kk in range(NS // GCH):
            g.wait()
            if kk + 1 < NS // GCH:
                g = pltpu.async_copy(
                    local_hbm.at[idx_v.at[pl.ds(IOFF + (kk + 1) * GCH, GCH)]],
                    bufs[(kk + 1) % 2], sem)
            pltpu.sync_copy(bufs[kk % 2],
                            teb_hbm.at[pl.ds(bl * NS + kk * GCH, GCH)])

        def _runs(k, _):
            v = idx_v[pl.ds(IOFF + k * 16, 16)]
            vp = idx_v[pl.ds(IOFF - 1 + k * 16, 16)]
            vn = idx_v[pl.ds(IOFF + 1 + k * 16, 16)]
            first = v != vp
            last = v != vn
            gp = k.astype(jnp.float32) * 16.0 + io16f
            plsc.store_scatter(cb2, [v], gp, mask=first)
            plsc.store_scatter(cb1, [v], gp + 1.0, mask=last)
            return _
        lax.fori_loop(0, NS // 16, _runs, None)

        def _counts(k, _):
            v = idx_v[pl.ds(IOFF + k * 16, 16)]
            vn = idx_v[pl.ds(IOFF + 1 + k * 16, 16)]
            last = v != vn
            gp = k.astype(jnp.float32) * 16.0 + io16f
            firstpos = plsc.load_gather(cb2, [v])
            plsc.store_scatter(cb1, [v], gp + 1.0 - firstpos, mask=last)
            return _
        lax.fori_loop(0, NS // 16, _counts, None)

        pltpu.sync_copy(cb1, cnt_hbm.at[pl.ds(bl * NPAD, NPAD)])

        def _restore(k, _):
            v = idx_v[pl.ds(IOFF + k * 16, 16)]
            plsc.store_scatter(cb1, [v], z16f)
            plsc.store_scatter(cb2, [v], z16f)
            return _
        lax.fori_loop(0, NS // 16, _restore, None)

        adj_dma.wait()

        def _adj_clr(k, _):
            er = er_v[pl.ds(k * 16, 16)]
            ec = ec_v[pl.ds(k * 16, 16)]
            plsc.store_scatter(adjbuf, [er, ec], z16f)
            return _
        lax.fori_loop(0, ES // 16, _adj_clr, None)
        return _

    lax.fori_loop(0, npw, _node, None)


def _sc_gather_scatter(local, nidx_flat, ee_flat, b0, hb):
    mesh = plsc.VectorSubcoreMesh(
        core_axis_name="c", subcore_axis_name="s",
        num_cores=NC, num_subcores=NSUB)
    fn = functools.partial(
        pl.kernel, functools.partial(_sc_body, b0, hb // NW),
        out_type=[
            jax.ShapeDtypeStruct((hb * NS, DW), jnp.int32),
            jax.ShapeDtypeStruct((hb * NS, NS), jnp.float32),
            jax.ShapeDtypeStruct((hb * NPAD,), jnp.float32),
        ],
        mesh=mesh,
        compiler_params=pltpu.CompilerParams(needs_layout_passes=False),
        scratch_types=[
            pltpu.VMEM((IOFF + NS + 16,), jnp.int32),
            pltpu.VMEM((GCH, DW), jnp.int32),
            pltpu.VMEM((GCH, DW), jnp.int32),
            pltpu.VMEM((ES,), jnp.int32),
            pltpu.VMEM((ES,), jnp.int32),
            pltpu.VMEM((NS, NS), jnp.float32),
            pltpu.VMEM((NPAD,), jnp.float32),
            pltpu.VMEM((NPAD,), jnp.float32),
            pltpu.SemaphoreType.DMA,
            pltpu.SemaphoreType.DMA,
        ],
    )()
    return fn(local, nidx_flat, ee_flat)


def _fdiff_body(c_ref, d_ref, out_ref, acc):
    k = pl.program_id(0)

    @pl.when(k == 0)
    def _():
        acc[...] = jnp.zeros_like(acc)

    acc[...] += lax.dot_general(
        c_ref[...], d_ref[...], (((1,), (0,)), ((), ())),
        preferred_element_type=jnp.float32,
        precision=lax.Precision.HIGHEST)

    @pl.when(k == pl.num_programs(0) - 1)
    def _():
        s = acc[...]
        fd = jnp.sqrt(jnp.sum(s * s, axis=1, keepdims=True)) / NS
        out_ref[...] = jnp.broadcast_to(fd, (out_ref.shape[0], 128))


def _fdiff(cnt, dtab):
    hb = cnt.shape[0]
    return pl.pallas_call(
        _fdiff_body,
        grid=(NPAD // KBLK,),
        in_specs=[
            pl.BlockSpec((hb, KBLK), lambda k: (0, k)),
            pl.BlockSpec((KBLK, D), lambda k: (k, 0)),
        ],
        out_specs=pl.BlockSpec((hb, 128), lambda k: (0, 0)),
        out_shape=jax.ShapeDtypeStruct((hb, 128), jnp.float32),
        scratch_shapes=[pltpu.VMEM((hb, D), jnp.float32)],
    )(cnt, dtab)


def _sdiff_body(teb_ref, adj_ref, u_ref, out_ref):
    dn = (((1,), (1,)), ((), ()))
    for mn in range(MN):
        tw = teb_ref[pl.ds(mn * NS, NS), :]
        lo = lax.bitcast_convert_type(
            (tw & 0xFFFF).astype(jnp.uint16), jnp.bfloat16)
        hi = lax.bitcast_convert_type(
            (lax.shift_right_logical(tw, 16)).astype(jnp.uint16), jnp.bfloat16)
        sim = (lax.dot_general(lo, lo, dn, preferred_element_type=jnp.float32)
               + lax.dot_general(hi, hi, dn, preferred_element_type=jnp.float32))
        u = u_ref[mn]
        g = -jnp.log(-jnp.log(u + 1e-9) + 1e-9)
        z = sim - THRESHOLD + g
        e = jnp.exp(z)
        p = e / jnp.sum(e, axis=1, keepdims=True)
        df = adj_ref[pl.ds(mn * NS, NS), :] - p
        sr = jnp.sqrt(jnp.sum(df * df, axis=1, keepdims=True))
        sdiff = jnp.sum(sr) / NS
        out_ref[pl.ds(mn, 1)] = jnp.full((1, 1, 128), sdiff, jnp.float32)


def _sdiff(teb, adj2, gumbel):
    hb = teb.shape[0] // NS
    return pl.pallas_call(
        _sdiff_body,
        grid=(hb // MN,),
        in_specs=[
            pl.BlockSpec((MN * NS, DW), lambda i: (i, 0)),
            pl.BlockSpec((MN * NS, NS), lambda i: (i, 0)),
            pl.BlockSpec((MN, NS, NS), lambda i: (i, 0, 0)),
        ],
        out_specs=pl.BlockSpec((MN, 1, 128), lambda i: (i, 0, 0)),
        out_shape=jax.ShapeDtypeStruct((hb, 1, 128), jnp.float32),
    )(teb, adj2, gumbel)


def _final_body(sd_ref, fd_ref, y_ref, score_ref, loss_ref):
    sraw = sd_ref[...] + fd_ref[...]
    mn = jnp.min(sraw)
    mx = jnp.max(sraw)
    sc = (sraw - mn) / (mx - mn)
    y = y_ref[...]
    logp = jnp.maximum(jnp.log(sc), -100.0)
    log1mp = jnp.maximum(jnp.log(1.0 - sc), -100.0)
    bce = -jnp.mean(y * logp + (1.0 - y) * log1mp)
    score_ref[...] = sc
    loss_ref[...] = jnp.full((8, 128), bce / B, jnp.float32)


def _finalize(sdcol, fdcol, ytab):
    return pl.pallas_call(
        _final_body,
        out_shape=[
            jax.ShapeDtypeStruct((B, 128), jnp.float32),
            jax.ShapeDtypeStruct((8, 128), jnp.float32),
        ],
    )(sdcol, fdcol, ytab)


def kernel(feature, text_embedding, virtual, gumbel_noise, train_nodes,
           neighbor_idx, ego_edges, train_label):
    local, dtab = _normalize(feature, text_embedding, virtual)
    nidx_flat = neighbor_idx.reshape(B * NS)
    ee_flat = ego_edges.reshape(B * 2 * ES)
    hb = B // NCHUNK
    sds, fds = [], []
    for c in range(NCHUNK):
        teb, adjf, cnt = _sc_gather_scatter(
            local, nidx_flat, ee_flat, c * hb, hb)
        sd3 = _sdiff(teb, adjf, gumbel_noise[c * hb:(c + 1) * hb])
        fds.append(_fdiff(cnt.reshape(hb, NPAD), dtab))
        sds.append(sd3.reshape(hb, 128))
    sdcol = jnp.concatenate(sds, axis=0)
    fdcol = jnp.concatenate(fds, axis=0)
    ytab = jnp.broadcast_to(
        train_label.astype(jnp.float32)[:, None], (B, 128))
    scoreb, lossb = _finalize(sdcol, fdcol, ytab)
    return scoreb[:, 0], lossb[0, 0]
